# Initial kernel scaffold; baseline (speedup 1.0000x reference)
import jax
import jax.numpy as jnp
from jax.experimental import pallas as pl


def kernel(x, edge_index, W_l, W_r, b):
    raise NotImplementedError("write your pallas kernel here")


# speedup vs baseline: 3.4923x; 3.4923x over previous
import functools

import jax
import jax.numpy as jnp
from jax import lax
from jax.experimental import pallas as pl
from jax.experimental.pallas import tpu as pltpu
from jax.experimental.pallas import tpu_sc as plsc

N_NODES = 10000
N_EDGES = 320000
D = 128

NC = 2
NS = 16
NW = NC * NS

C = 128
E_PER_TILE = 10240
N_CHUNKS = E_PER_TILE // C
E_PAD = E_PER_TILE * NW
N_ACC = 10240
ROWS_PER_TILE = N_ACC // NS
CPY = 128
N_CPY = ROWS_PER_TILE // CPY


def _sc_body(x_hbm, src_hbm, dst_hbm, psum_hbm, pcnt_hbm, src_v, dst_v,
             rows_v, ones_v, acc_sh, sem):
    cid = lax.axis_index("c")
    sid = lax.axis_index("s")
    wid = cid * NS + sid
    row0 = sid * ROWS_PER_TILE
    tile_base = wid * E_PER_TILE

    def _one_row(i, _):
        for j in range(D // 16):
            ones_v[i, pl.ds(j * 16, 16)] = jnp.ones((16,), jnp.float32)
        return 0

    lax.fori_loop(0, C, _one_row, 0)

    def _zero_row(i, _):
        for j in range(D // 16):
            rows_v[i, pl.ds(j * 16, 16)] = jnp.zeros((16,), jnp.float32)
        return 0

    def _zero_own_rows():
        for j in range(N_CPY):
            pltpu.sync_copy(rows_v, acc_sh.at[pl.ds(row0 + j * CPY, CPY), :])

    lax.fori_loop(0, CPY, _zero_row, 0)
    _zero_own_rows()
    plsc.subcore_barrier()

    def _chunk(i, _):
        base = tile_base + i * C
        pltpu.sync_copy(src_hbm.at[pl.ds(base, C)], src_v)
        pltpu.sync_copy(dst_hbm.at[pl.ds(base, C)], dst_v)
        pltpu.async_copy(x_hbm.at[src_v], rows_v, sem).wait()
        pltpu.sync_copy(rows_v, acc_sh.at[dst_v], add=True)
        return 0

    lax.fori_loop(0, N_CHUNKS, _chunk, 0)
    plsc.subcore_barrier()

    for j in range(N_CPY):
        r = row0 + j * CPY
        pltpu.sync_copy(acc_sh.at[pl.ds(r, CPY), :], rows_v)
        pltpu.sync_copy(rows_v, psum_hbm.at[cid, pl.ds(r, CPY), :])

    lax.fori_loop(0, CPY, _zero_row, 0)
    _zero_own_rows()
    plsc.subcore_barrier()

    def _cnt_chunk(i, _):
        base = tile_base + i * C
        pltpu.sync_copy(dst_hbm.at[pl.ds(base, C)], dst_v)
        pltpu.sync_copy(ones_v, acc_sh.at[dst_v], add=True)
        return 0

    lax.fori_loop(0, N_CHUNKS, _cnt_chunk, 0)
    plsc.subcore_barrier()

    for j in range(N_CPY):
        r = row0 + j * CPY
        pltpu.sync_copy(acc_sh.at[pl.ds(r, CPY), :], rows_v)
        pltpu.sync_copy(rows_v, pcnt_hbm.at[cid, pl.ds(r, CPY), :])


_sc_aggregate = functools.partial(
    pl.kernel,
    out_type=(
        jax.ShapeDtypeStruct((NC, N_ACC, D), jnp.float32),
        jax.ShapeDtypeStruct((NC, N_ACC, D), jnp.float32),
    ),
    mesh=plsc.VectorSubcoreMesh(
        core_axis_name="c", subcore_axis_name="s", num_cores=NC, num_subcores=NS
    ),
    scratch_types=[
        pltpu.VMEM((C,), jnp.int32),
        pltpu.VMEM((C,), jnp.int32),
        pltpu.VMEM((CPY, D), jnp.float32),
        pltpu.VMEM((C, D), jnp.float32),
        pltpu.VMEM_SHARED((N_ACC, D), jnp.float32),
        pltpu.SemaphoreType.DMA,
    ],
)(_sc_body)


BLK = 1000
N_BLK = N_NODES // BLK


def _tc_body(p_ref, c_ref, x_ref, wl_ref, wr_ref, b_ref, o_ref):
    p = p_ref[...]
    s = p[0] + p[1]
    c = c_ref[...]
    cnt = (c[0] + c[1])[:, 0:1]
    mean = s / jnp.maximum(cnt, 1.0)
    h = (
        jnp.dot(mean, wl_ref[...], preferred_element_type=jnp.float32)
        + jnp.dot(x_ref[...], wr_ref[...], preferred_element_type=jnp.float32)
        + b_ref[...]
    )
    h = jnp.maximum(h, 0.0)
    m = jnp.max(h, axis=1, keepdims=True)
    lse = jnp.log(jnp.sum(jnp.exp(h - m), axis=1, keepdims=True)) + m
    o_ref[...] = h - lse


def _tc_dense(psum, pcnt, x, wlT, wrT, b2):
    return pl.pallas_call(
        _tc_body,
        grid=(N_BLK,),
        in_specs=[
            pl.BlockSpec((NC, BLK, D), lambda i: (0, i, 0)),
            pl.BlockSpec((NC, BLK, D), lambda i: (0, i, 0)),
            pl.BlockSpec((BLK, D), lambda i: (i, 0)),
            pl.BlockSpec((D, D), lambda i: (0, 0)),
            pl.BlockSpec((D, D), lambda i: (0, 0)),
            pl.BlockSpec((1, D), lambda i: (0, 0)),
        ],
        out_specs=pl.BlockSpec((BLK, D), lambda i: (i, 0)),
        out_shape=jax.ShapeDtypeStruct((N_NODES, D), jnp.float32),
    )(psum, pcnt, x, wlT, wrT, b2)


def kernel(x, edge_index, W_l, W_r, b):
    src = edge_index[0].astype(jnp.int32)
    dst = edge_index[1].astype(jnp.int32)
    pad = E_PAD - N_EDGES
    src_p = jnp.concatenate([src, jnp.zeros((pad,), jnp.int32)])
    dst_p = jnp.concatenate([dst, jnp.full((pad,), N_NODES, jnp.int32)])
    psum, pcnt = _sc_aggregate(x, src_p, dst_p)
    return _tc_dense(psum, pcnt, x, W_l.T, W_r.T, b.reshape(1, D))


# speedup vs baseline: 4.5299x; 1.2971x over previous
import functools

import jax
import jax.numpy as jnp
from jax import lax
from jax.experimental import pallas as pl
from jax.experimental.pallas import tpu as pltpu
from jax.experimental.pallas import tpu_sc as plsc

N_NODES = 10000
N_EDGES = 320000
D = 128

NC = 2
NS = 16
NW = NC * NS

C = 128
E_PER_TILE = 10240
N_CHUNKS = E_PER_TILE // C
E_PAD = E_PER_TILE * NW
N_ACC = 10240
ROWS_PER_TILE = N_ACC // NS
CPY = 128
N_CPY = ROWS_PER_TILE // CPY


H = N_CHUNKS // 2


def _sc_body(x_hbm, src_hbm, dst_hbm, psum_hbm, pcnt_hbm, src_a, dst_a,
             r0, r1, acc_sh, sem0, sem1):
    cid = lax.axis_index("c")
    sid = lax.axis_index("s")
    wid = cid * NS + sid
    row0 = sid * ROWS_PER_TILE
    chunk0 = wid * N_CHUNKS

    def _fill_rows(buf, val):
        def _row(i, _):
            for j in range(D // 16):
                buf[i, pl.ds(j * 16, 16)] = jnp.full((16,), val, jnp.float32)
            return 0

        lax.fori_loop(0, CPY, _row, 0)

    def _zero_own_rows(buf):
        for j in range(N_CPY):
            pltpu.sync_copy(buf, acc_sh.at[pl.ds(row0 + j * CPY, CPY), :])

    _fill_rows(r0, 0.0)
    _zero_own_rows(r0)
    plsc.subcore_barrier()

    for h in range(2):
        pltpu.sync_copy(src_hbm.at[pl.ds(chunk0 + h * H, H + 8), :], src_a)
        pltpu.sync_copy(dst_hbm.at[pl.ds(chunk0 + h * H, H), :], dst_a)
        pltpu.async_copy(x_hbm.at[src_a.at[0]], r0, sem0)

        def _pair(g, _):
            i = 2 * g
            pltpu.async_copy(x_hbm.at[src_a.at[i + 1]], r1, sem1)
            pltpu.make_async_copy(x_hbm.at[src_a.at[i]], r0, sem0).wait()
            pltpu.sync_copy(r0, acc_sh.at[dst_a.at[i]], add=True)
            pltpu.async_copy(x_hbm.at[src_a.at[i + 2]], r0, sem0)
            pltpu.make_async_copy(x_hbm.at[src_a.at[i + 1]], r1, sem1).wait()
            pltpu.sync_copy(r1, acc_sh.at[dst_a.at[i + 1]], add=True)
            return 0

        lax.fori_loop(0, H // 2, _pair, 0)
        pltpu.make_async_copy(x_hbm.at[src_a.at[0]], r0, sem0).wait()

    plsc.subcore_barrier()

    for j in range(N_CPY):
        r = row0 + j * CPY
        pltpu.sync_copy(acc_sh.at[pl.ds(r, CPY), :], r0)
        pltpu.sync_copy(r0, psum_hbm.at[cid, pl.ds(r, CPY), :])

    _fill_rows(r1, 0.0)
    _zero_own_rows(r1)
    _fill_rows(r0, 1.0)
    plsc.subcore_barrier()

    for h in range(2):
        pltpu.sync_copy(dst_hbm.at[pl.ds(chunk0 + h * H, H), :], dst_a)

        def _cnt_chunk(i, _):
            pltpu.sync_copy(r0, acc_sh.at[dst_a.at[i]], add=True)
            return 0

        lax.fori_loop(0, H, _cnt_chunk, 0)

    plsc.subcore_barrier()

    for j in range(N_CPY):
        r = row0 + j * CPY
        pltpu.sync_copy(acc_sh.at[pl.ds(r, CPY), :], r1)
        pltpu.sync_copy(r1, pcnt_hbm.at[cid, pl.ds(r, CPY), :])


_sc_aggregate = functools.partial(
    pl.kernel,
    out_type=(
        jax.ShapeDtypeStruct((NC, N_ACC, D), jnp.float32),
        jax.ShapeDtypeStruct((NC, N_ACC, D), jnp.float32),
    ),
    mesh=plsc.VectorSubcoreMesh(
        core_axis_name="c", subcore_axis_name="s", num_cores=NC, num_subcores=NS
    ),
    scratch_types=[
        pltpu.VMEM((H + 8, C), jnp.int32),
        pltpu.VMEM((H, C), jnp.int32),
        pltpu.VMEM((CPY, D), jnp.float32),
        pltpu.VMEM((CPY, D), jnp.float32),
        pltpu.VMEM_SHARED((N_ACC, D), jnp.float32),
        pltpu.SemaphoreType.DMA,
        pltpu.SemaphoreType.DMA,
    ],
)(_sc_body)


BLK = 1000
N_BLK = N_NODES // BLK


def _tc_body(p_ref, c_ref, x_ref, wl_ref, wr_ref, b_ref, o_ref):
    p = p_ref[...]
    s = p[0] + p[1]
    c = c_ref[...]
    cnt = (c[0] + c[1])[:, 0:1]
    mean = s / jnp.maximum(cnt, 1.0)
    h = (
        jnp.dot(mean, wl_ref[...], preferred_element_type=jnp.float32)
        + jnp.dot(x_ref[...], wr_ref[...], preferred_element_type=jnp.float32)
        + b_ref[...]
    )
    h = jnp.maximum(h, 0.0)
    m = jnp.max(h, axis=1, keepdims=True)
    lse = jnp.log(jnp.sum(jnp.exp(h - m), axis=1, keepdims=True)) + m
    o_ref[...] = h - lse


def _tc_dense(psum, pcnt, x, wlT, wrT, b2):
    return pl.pallas_call(
        _tc_body,
        grid=(N_BLK,),
        in_specs=[
            pl.BlockSpec((NC, BLK, D), lambda i: (0, i, 0)),
            pl.BlockSpec((NC, BLK, D), lambda i: (0, i, 0)),
            pl.BlockSpec((BLK, D), lambda i: (i, 0)),
            pl.BlockSpec((D, D), lambda i: (0, 0)),
            pl.BlockSpec((D, D), lambda i: (0, 0)),
            pl.BlockSpec((1, D), lambda i: (0, 0)),
        ],
        out_specs=pl.BlockSpec((BLK, D), lambda i: (i, 0)),
        out_shape=jax.ShapeDtypeStruct((N_NODES, D), jnp.float32),
    )(psum, pcnt, x, wlT, wrT, b2)


def kernel(x, edge_index, W_l, W_r, b):
    src = edge_index[0].astype(jnp.int32)
    dst = edge_index[1].astype(jnp.int32)
    src_p = jnp.concatenate(
        [src, jnp.zeros((E_PAD + 8 * C - N_EDGES,), jnp.int32)]
    ).reshape(NW * N_CHUNKS + 8, C)
    dst_p = jnp.concatenate(
        [dst, jnp.full((E_PAD - N_EDGES,), N_NODES, jnp.int32)]
    ).reshape(NW * N_CHUNKS, C)
    psum, pcnt = _sc_aggregate(x, src_p, dst_p)
    return _tc_dense(psum, pcnt, x, W_l.T, W_r.T, b.reshape(1, D))


# speedup vs baseline: 4.5337x; 1.0008x over previous
import functools

import jax
import jax.numpy as jnp
from jax import lax
from jax.experimental import pallas as pl
from jax.experimental.pallas import tpu as pltpu
from jax.experimental.pallas import tpu_sc as plsc

N_NODES = 10000
N_EDGES = 320000
D = 128

NC = 2
NS = 16
NW = NC * NS

C = 128
E_PER_TILE = 10240
N_CHUNKS = E_PER_TILE // C
E_PAD = E_PER_TILE * NW
N_ACC = 10240
ROWS_PER_TILE = N_ACC // NS
CPY = 128
N_CPY = ROWS_PER_TILE // CPY


H = N_CHUNKS // 2


def _sc_body(x_hbm, src_hbm, dst_hbm, psum_hbm, pcnt_hbm, src_a, dst_a,
             r0, r1, acc_sh, sem0, sem1, sem2, sem3):
    cid = lax.axis_index("c")
    sid = lax.axis_index("s")
    wid = cid * NS + sid
    row0 = sid * ROWS_PER_TILE
    chunk0 = wid * N_CHUNKS

    HC = C // 2

    def _gather_start(i, buf, sa, sb):
        pltpu.async_copy(x_hbm.at[src_a.at[i, pl.ds(0, HC)]],
                         buf.at[pl.ds(0, HC), :], sa)
        pltpu.async_copy(x_hbm.at[src_a.at[i, pl.ds(HC, HC)]],
                         buf.at[pl.ds(HC, HC), :], sb)

    def _gather_wait(i, buf, sa, sb):
        pltpu.make_async_copy(x_hbm.at[src_a.at[i, pl.ds(0, HC)]],
                              buf.at[pl.ds(0, HC), :], sa).wait()
        pltpu.make_async_copy(x_hbm.at[src_a.at[i, pl.ds(HC, HC)]],
                              buf.at[pl.ds(HC, HC), :], sb).wait()

    def _fill_rows(buf, val):
        def _row(i, _):
            for j in range(D // 16):
                buf[i, pl.ds(j * 16, 16)] = jnp.full((16,), val, jnp.float32)
            return 0

        lax.fori_loop(0, CPY, _row, 0)

    def _zero_own_rows(buf):
        for j in range(N_CPY):
            pltpu.sync_copy(buf, acc_sh.at[pl.ds(row0 + j * CPY, CPY), :])

    _fill_rows(r0, 0.0)
    _zero_own_rows(r0)
    plsc.subcore_barrier()

    for h in range(2):
        pltpu.sync_copy(src_hbm.at[pl.ds(chunk0 + h * H, H + 8), :], src_a)
        pltpu.sync_copy(dst_hbm.at[pl.ds(chunk0 + h * H, H), :], dst_a)
        _gather_start(0, r0, sem0, sem1)

        def _pair(g, _):
            i = 2 * g
            _gather_start(i + 1, r1, sem2, sem3)
            _gather_wait(i, r0, sem0, sem1)
            pltpu.sync_copy(r0, acc_sh.at[dst_a.at[i]], add=True)
            _gather_start(i + 2, r0, sem0, sem1)
            _gather_wait(i + 1, r1, sem2, sem3)
            pltpu.sync_copy(r1, acc_sh.at[dst_a.at[i + 1]], add=True)
            return 0

        lax.fori_loop(0, H // 2, _pair, 0)
        _gather_wait(0, r0, sem0, sem1)

    plsc.subcore_barrier()

    for j in range(N_CPY):
        r = row0 + j * CPY
        pltpu.sync_copy(acc_sh.at[pl.ds(r, CPY), :], r0)
        pltpu.sync_copy(r0, psum_hbm.at[cid, pl.ds(r, CPY), :])

    _fill_rows(r1, 0.0)
    _zero_own_rows(r1)
    _fill_rows(r0, 1.0)
    plsc.subcore_barrier()

    for h in range(2):
        pltpu.sync_copy(dst_hbm.at[pl.ds(chunk0 + h * H, H), :], dst_a)

        def _cnt_chunk(i, _):
            pltpu.sync_copy(r0, acc_sh.at[dst_a.at[i]], add=True)
            return 0

        lax.fori_loop(0, H, _cnt_chunk, 0)

    plsc.subcore_barrier()

    for j in range(N_CPY):
        r = row0 + j * CPY
        pltpu.sync_copy(acc_sh.at[pl.ds(r, CPY), :], r1)
        pltpu.sync_copy(r1, pcnt_hbm.at[cid, pl.ds(r, CPY), :])


_sc_aggregate = functools.partial(
    pl.kernel,
    out_type=(
        jax.ShapeDtypeStruct((NC, N_ACC, D), jnp.float32),
        jax.ShapeDtypeStruct((NC, N_ACC, D), jnp.float32),
    ),
    mesh=plsc.VectorSubcoreMesh(
        core_axis_name="c", subcore_axis_name="s", num_cores=NC, num_subcores=NS
    ),
    scratch_types=[
        pltpu.VMEM((H + 8, C), jnp.int32),
        pltpu.VMEM((H, C), jnp.int32),
        pltpu.VMEM((CPY, D), jnp.float32),
        pltpu.VMEM((CPY, D), jnp.float32),
        pltpu.VMEM_SHARED((N_ACC, D), jnp.float32),
        pltpu.SemaphoreType.DMA,
        pltpu.SemaphoreType.DMA,
        pltpu.SemaphoreType.DMA,
        pltpu.SemaphoreType.DMA,
    ],
)(_sc_body)


BLK = 1000
N_BLK = N_NODES // BLK


def _tc_body(p_ref, c_ref, x_ref, wl_ref, wr_ref, b_ref, o_ref):
    p = p_ref[...]
    s = p[0] + p[1]
    c = c_ref[...]
    cnt = (c[0] + c[1])[:, 0:1]
    mean = s / jnp.maximum(cnt, 1.0)
    h = (
        jnp.dot(mean, wl_ref[...], preferred_element_type=jnp.float32)
        + jnp.dot(x_ref[...], wr_ref[...], preferred_element_type=jnp.float32)
        + b_ref[...]
    )
    h = jnp.maximum(h, 0.0)
    m = jnp.max(h, axis=1, keepdims=True)
    lse = jnp.log(jnp.sum(jnp.exp(h - m), axis=1, keepdims=True)) + m
    o_ref[...] = h - lse


def _tc_dense(psum, pcnt, x, wlT, wrT, b2):
    return pl.pallas_call(
        _tc_body,
        grid=(N_BLK,),
        in_specs=[
            pl.BlockSpec((NC, BLK, D), lambda i: (0, i, 0)),
            pl.BlockSpec((NC, BLK, D), lambda i: (0, i, 0)),
            pl.BlockSpec((BLK, D), lambda i: (i, 0)),
            pl.BlockSpec((D, D), lambda i: (0, 0)),
            pl.BlockSpec((D, D), lambda i: (0, 0)),
            pl.BlockSpec((1, D), lambda i: (0, 0)),
        ],
        out_specs=pl.BlockSpec((BLK, D), lambda i: (i, 0)),
        out_shape=jax.ShapeDtypeStruct((N_NODES, D), jnp.float32),
    )(psum, pcnt, x, wlT, wrT, b2)


def kernel(x, edge_index, W_l, W_r, b):
    src = edge_index[0].astype(jnp.int32)
    dst = edge_index[1].astype(jnp.int32)
    src_p = jnp.concatenate(
        [src, jnp.zeros((E_PAD + 8 * C - N_EDGES,), jnp.int32)]
    ).reshape(NW * N_CHUNKS + 8, C)
    dst_p = jnp.concatenate(
        [dst, jnp.full((E_PAD - N_EDGES,), N_NODES, jnp.int32)]
    ).reshape(NW * N_CHUNKS, C)
    psum, pcnt = _sc_aggregate(x, src_p, dst_p)
    return _tc_dense(psum, pcnt, x, W_l.T, W_r.T, b.reshape(1, D))


# speedup vs baseline: 4.5434x; 1.0022x over previous
import functools

import jax
import jax.numpy as jnp
from jax import lax
from jax.experimental import pallas as pl
from jax.experimental.pallas import tpu as pltpu
from jax.experimental.pallas import tpu_sc as plsc

N_NODES = 10000
N_EDGES = 320000
D = 128

NC = 2
NS = 16
NW = NC * NS

C = 128
E_PER_TILE = 10240
N_CHUNKS = E_PER_TILE // C
E_PAD = E_PER_TILE * NW
N_ACC = 10240
ROWS_PER_TILE = N_ACC // NS
CPY = 128
N_CPY = ROWS_PER_TILE // CPY


H = N_CHUNKS // 2


def _sc_body(x_hbm, src_hbm, dst_hbm, psum_hbm, pcnt_hbm, src_a, dst_a,
             r0, r1, acc_sh, sem0, sem1, sem2, sem3):
    cid = lax.axis_index("c")
    sid = lax.axis_index("s")
    wid = cid * NS + sid
    row0 = sid * ROWS_PER_TILE
    chunk0 = wid * N_CHUNKS

    HC = C // 2

    def _gather_start(i, buf, sa, sb):
        pltpu.async_copy(x_hbm.at[src_a.at[i, pl.ds(0, HC)]],
                         buf.at[pl.ds(0, HC), :], sa)
        pltpu.async_copy(x_hbm.at[src_a.at[i, pl.ds(HC, HC)]],
                         buf.at[pl.ds(HC, HC), :], sb)

    def _gather_wait(i, buf, sa, sb):
        pltpu.make_async_copy(x_hbm.at[src_a.at[i, pl.ds(0, HC)]],
                              buf.at[pl.ds(0, HC), :], sa).wait()
        pltpu.make_async_copy(x_hbm.at[src_a.at[i, pl.ds(HC, HC)]],
                              buf.at[pl.ds(HC, HC), :], sb).wait()

    def _fill_rows(buf, val):
        def _row(i, _):
            for j in range(D // 16):
                buf[i, pl.ds(j * 16, 16)] = jnp.full((16,), val, jnp.float32)
            return 0

        lax.fori_loop(0, CPY, _row, 0)

    def _zero_own_rows(buf):
        for j in range(N_CPY):
            pltpu.sync_copy(buf, acc_sh.at[pl.ds(row0 + j * CPY, CPY), :])

    _fill_rows(r0, 0.0)
    _zero_own_rows(r0)
    plsc.subcore_barrier()

    for h in range(2):
        pltpu.sync_copy(src_hbm.at[pl.ds(chunk0 + h * H, H + 8), :], src_a)
        pltpu.sync_copy(dst_hbm.at[pl.ds(chunk0 + h * H, H), :], dst_a)
        _gather_start(0, r0, sem0, sem1)

        def _pair(g, _):
            i = 2 * g
            _gather_start(i + 1, r1, sem2, sem3)
            _gather_wait(i, r0, sem0, sem1)
            pltpu.sync_copy(r0, acc_sh.at[dst_a.at[i]], add=True)
            _gather_start(i + 2, r0, sem0, sem1)
            _gather_wait(i + 1, r1, sem2, sem3)
            pltpu.sync_copy(r1, acc_sh.at[dst_a.at[i + 1]], add=True)
            return 0

        lax.fori_loop(0, H // 2, _pair, 0)
        _gather_wait(0, r0, sem0, sem1)

    plsc.subcore_barrier()

    pltpu.sync_copy(acc_sh.at[pl.ds(row0, ROWS_PER_TILE), :],
                    psum_hbm.at[cid, pl.ds(row0, ROWS_PER_TILE), :])

    _fill_rows(r1, 0.0)
    _zero_own_rows(r1)
    _fill_rows(r0, 1.0)
    plsc.subcore_barrier()

    for h in range(2):
        pltpu.sync_copy(dst_hbm.at[pl.ds(chunk0 + h * H, H), :], dst_a)

        def _cnt_chunk(i, _):
            pltpu.sync_copy(r0, acc_sh.at[dst_a.at[i]], add=True)
            return 0

        lax.fori_loop(0, H, _cnt_chunk, 0)

    plsc.subcore_barrier()

    pltpu.sync_copy(acc_sh.at[pl.ds(row0, ROWS_PER_TILE), :],
                    pcnt_hbm.at[cid, pl.ds(row0, ROWS_PER_TILE), :])


_sc_aggregate = functools.partial(
    pl.kernel,
    out_type=(
        jax.ShapeDtypeStruct((NC, N_ACC, D), jnp.float32),
        jax.ShapeDtypeStruct((NC, N_ACC, D), jnp.float32),
    ),
    mesh=plsc.VectorSubcoreMesh(
        core_axis_name="c", subcore_axis_name="s", num_cores=NC, num_subcores=NS
    ),
    scratch_types=[
        pltpu.VMEM((H + 8, C), jnp.int32),
        pltpu.VMEM((H, C), jnp.int32),
        pltpu.VMEM((CPY, D), jnp.float32),
        pltpu.VMEM((CPY, D), jnp.float32),
        pltpu.VMEM_SHARED((N_ACC, D), jnp.float32),
        pltpu.SemaphoreType.DMA,
        pltpu.SemaphoreType.DMA,
        pltpu.SemaphoreType.DMA,
        pltpu.SemaphoreType.DMA,
    ],
)(_sc_body)


BLK = 1000
N_BLK = N_NODES // BLK


def _tc_body(p_ref, c_ref, x_ref, wl_ref, wr_ref, b_ref, o_ref):
    p = p_ref[...]
    s = p[0] + p[1]
    c = c_ref[...]
    cnt = (c[0] + c[1])[:, 0:1]
    mean = s / jnp.maximum(cnt, 1.0)
    h = (
        jnp.dot(mean, wl_ref[...], preferred_element_type=jnp.float32)
        + jnp.dot(x_ref[...], wr_ref[...], preferred_element_type=jnp.float32)
        + b_ref[...]
    )
    h = jnp.maximum(h, 0.0)
    m = jnp.max(h, axis=1, keepdims=True)
    lse = jnp.log(jnp.sum(jnp.exp(h - m), axis=1, keepdims=True)) + m
    o_ref[...] = h - lse


def _tc_dense(psum, pcnt, x, wlT, wrT, b2):
    return pl.pallas_call(
        _tc_body,
        grid=(N_BLK,),
        in_specs=[
            pl.BlockSpec((NC, BLK, D), lambda i: (0, i, 0)),
            pl.BlockSpec((NC, BLK, D), lambda i: (0, i, 0)),
            pl.BlockSpec((BLK, D), lambda i: (i, 0)),
            pl.BlockSpec((D, D), lambda i: (0, 0)),
            pl.BlockSpec((D, D), lambda i: (0, 0)),
            pl.BlockSpec((1, D), lambda i: (0, 0)),
        ],
        out_specs=pl.BlockSpec((BLK, D), lambda i: (i, 0)),
        out_shape=jax.ShapeDtypeStruct((N_NODES, D), jnp.float32),
    )(psum, pcnt, x, wlT, wrT, b2)


def kernel(x, edge_index, W_l, W_r, b):
    src = edge_index[0].astype(jnp.int32)
    dst = edge_index[1].astype(jnp.int32)
    src_p = jnp.concatenate(
        [src, jnp.zeros((E_PAD + 8 * C - N_EDGES,), jnp.int32)]
    ).reshape(NW * N_CHUNKS + 8, C)
    dst_p = jnp.concatenate(
        [dst, jnp.full((E_PAD - N_EDGES,), N_NODES, jnp.int32)]
    ).reshape(NW * N_CHUNKS, C)
    psum, pcnt = _sc_aggregate(x, src_p, dst_p)
    return _tc_dense(psum, pcnt, x, W_l.T, W_r.T, b.reshape(1, D))


# speedup vs baseline: 4.5562x; 1.0028x over previous
import functools

import jax
import jax.numpy as jnp
from jax import lax
from jax.experimental import pallas as pl
from jax.experimental.pallas import tpu as pltpu
from jax.experimental.pallas import tpu_sc as plsc

N_NODES = 10000
N_EDGES = 320000
D = 128

NC = 2
NS = 16
NW = NC * NS

C = 128
E_PER_TILE = 10240
N_CHUNKS = E_PER_TILE // C
E_PAD = E_PER_TILE * NW
N_ACC = 10240
ROWS_PER_TILE = N_ACC // NS
CPY = 128
N_CPY = ROWS_PER_TILE // CPY


H = N_CHUNKS // 2


def _sc_body(x_hbm, src_hbm, dst_hbm, psum_hbm, pcnt_hbm, src_a, dst_a,
             r0, r1, acc_sh, sem0, sem1, sem2, sem3):
    cid = lax.axis_index("c")
    sid = lax.axis_index("s")
    wid = cid * NS + sid
    row0 = sid * ROWS_PER_TILE
    chunk0 = wid * N_CHUNKS

    HC = C // 2

    def _gather_start(i, buf, sa, sb):
        pltpu.async_copy(x_hbm.at[src_a.at[i, pl.ds(0, HC)]],
                         buf.at[pl.ds(0, HC), :], sa)
        pltpu.async_copy(x_hbm.at[src_a.at[i, pl.ds(HC, HC)]],
                         buf.at[pl.ds(HC, HC), :], sb)

    def _gather_wait(i, buf, sa, sb):
        pltpu.make_async_copy(x_hbm.at[src_a.at[i, pl.ds(0, HC)]],
                              buf.at[pl.ds(0, HC), :], sa).wait()
        pltpu.make_async_copy(x_hbm.at[src_a.at[i, pl.ds(HC, HC)]],
                              buf.at[pl.ds(HC, HC), :], sb).wait()

    def _fill_rows(buf, val):
        def _row(i, _):
            for j in range(D // 16):
                buf[i, pl.ds(j * 16, 16)] = jnp.full((16,), val, jnp.float32)
            return 0

        lax.fori_loop(0, CPY, _row, 0)

    def _zero_own_rows(buf):
        for j in range(N_CPY):
            pltpu.sync_copy(buf, acc_sh.at[pl.ds(row0 + j * CPY, CPY), :])

    _fill_rows(r0, 0.0)
    _zero_own_rows(r0)
    plsc.subcore_barrier()

    for h in range(2):
        pltpu.sync_copy(src_hbm.at[pl.ds(chunk0 + h * H, H + 8), :], src_a)
        pltpu.sync_copy(dst_hbm.at[pl.ds(chunk0 + h * H, H), :], dst_a)
        _gather_start(0, r0, sem0, sem1)

        def _pair(g, _):
            i = 2 * g
            _gather_start(i + 1, r1, sem2, sem3)
            _gather_wait(i, r0, sem0, sem1)
            pltpu.sync_copy(r0, acc_sh.at[dst_a.at[i]], add=True)
            _gather_start(i + 2, r0, sem0, sem1)
            _gather_wait(i + 1, r1, sem2, sem3)
            pltpu.sync_copy(r1, acc_sh.at[dst_a.at[i + 1]], add=True)
            return 0

        lax.fori_loop(0, H // 2, _pair, 0)
        _gather_wait(0, r0, sem0, sem1)

    plsc.subcore_barrier()

    pltpu.sync_copy(acc_sh.at[pl.ds(row0, ROWS_PER_TILE), :],
                    psum_hbm.at[cid, pl.ds(row0, ROWS_PER_TILE), :])

    _fill_rows(r1, 0.0)
    _zero_own_rows(r1)
    _fill_rows(r0, 1.0)
    plsc.subcore_barrier()

    for h in (1, 0):
        if h == 0:
            pltpu.sync_copy(dst_hbm.at[pl.ds(chunk0, H), :], dst_a)
        pltpu.async_copy(r0, acc_sh.at[dst_a.at[0]], sem0, add=True)

        def _cnt_pair(g, _):
            i = 2 * g
            pltpu.async_copy(r0, acc_sh.at[dst_a.at[i + 1]], sem1, add=True)
            pltpu.make_async_copy(r0, acc_sh.at[dst_a.at[i]], sem0).wait()
            pltpu.async_copy(r0, acc_sh.at[dst_a.at[i + 2]], sem0, add=True)
            pltpu.make_async_copy(r0, acc_sh.at[dst_a.at[i + 1]], sem1).wait()
            return 0

        lax.fori_loop(0, H // 2 - 1, _cnt_pair, 0)
        i = H - 2
        pltpu.async_copy(r0, acc_sh.at[dst_a.at[i + 1]], sem1, add=True)
        pltpu.make_async_copy(r0, acc_sh.at[dst_a.at[i]], sem0).wait()
        pltpu.make_async_copy(r0, acc_sh.at[dst_a.at[i + 1]], sem1).wait()

    plsc.subcore_barrier()

    pltpu.sync_copy(acc_sh.at[pl.ds(row0, ROWS_PER_TILE), :],
                    pcnt_hbm.at[cid, pl.ds(row0, ROWS_PER_TILE), :])


_sc_aggregate = functools.partial(
    pl.kernel,
    out_type=(
        jax.ShapeDtypeStruct((NC, N_ACC, D), jnp.float32),
        jax.ShapeDtypeStruct((NC, N_ACC, D), jnp.float32),
    ),
    mesh=plsc.VectorSubcoreMesh(
        core_axis_name="c", subcore_axis_name="s", num_cores=NC, num_subcores=NS
    ),
    scratch_types=[
        pltpu.VMEM((H + 8, C), jnp.int32),
        pltpu.VMEM((H, C), jnp.int32),
        pltpu.VMEM((CPY, D), jnp.float32),
        pltpu.VMEM((CPY, D), jnp.float32),
        pltpu.VMEM_SHARED((N_ACC, D), jnp.float32),
        pltpu.SemaphoreType.DMA,
        pltpu.SemaphoreType.DMA,
        pltpu.SemaphoreType.DMA,
        pltpu.SemaphoreType.DMA,
    ],
)(_sc_body)


BLK = 1000
N_BLK = N_NODES // BLK


def _tc_body(p_ref, c_ref, x_ref, wl_ref, wr_ref, b_ref, o_ref):
    p = p_ref[...]
    s = p[0] + p[1]
    c = c_ref[...]
    cnt = (c[0] + c[1])[:, 0:1]
    mean = s / jnp.maximum(cnt, 1.0)
    h = (
        jnp.dot(mean, wl_ref[...], preferred_element_type=jnp.float32)
        + jnp.dot(x_ref[...], wr_ref[...], preferred_element_type=jnp.float32)
        + b_ref[...]
    )
    h = jnp.maximum(h, 0.0)
    m = jnp.max(h, axis=1, keepdims=True)
    lse = jnp.log(jnp.sum(jnp.exp(h - m), axis=1, keepdims=True)) + m
    o_ref[...] = h - lse


def _tc_dense(psum, pcnt, x, wlT, wrT, b2):
    return pl.pallas_call(
        _tc_body,
        grid=(N_BLK,),
        in_specs=[
            pl.BlockSpec((NC, BLK, D), lambda i: (0, i, 0)),
            pl.BlockSpec((NC, BLK, D), lambda i: (0, i, 0)),
            pl.BlockSpec((BLK, D), lambda i: (i, 0)),
            pl.BlockSpec((D, D), lambda i: (0, 0)),
            pl.BlockSpec((D, D), lambda i: (0, 0)),
            pl.BlockSpec((1, D), lambda i: (0, 0)),
        ],
        out_specs=pl.BlockSpec((BLK, D), lambda i: (i, 0)),
        out_shape=jax.ShapeDtypeStruct((N_NODES, D), jnp.float32),
    )(psum, pcnt, x, wlT, wrT, b2)


def kernel(x, edge_index, W_l, W_r, b):
    src = edge_index[0].astype(jnp.int32)
    dst = edge_index[1].astype(jnp.int32)
    src_p = jnp.concatenate(
        [src, jnp.zeros((E_PAD + 8 * C - N_EDGES,), jnp.int32)]
    ).reshape(NW * N_CHUNKS + 8, C)
    dst_p = jnp.concatenate(
        [dst, jnp.full((E_PAD - N_EDGES,), N_NODES, jnp.int32)]
    ).reshape(NW * N_CHUNKS, C)
    psum, pcnt = _sc_aggregate(x, src_p, dst_p)
    return _tc_dense(psum, pcnt, x, W_l.T, W_r.T, b.reshape(1, D))
